# Initial kernel scaffold; baseline (speedup 1.0000x reference)
import jax
import jax.numpy as jnp
from jax.experimental import pallas as pl


def kernel(inputs, centroids):
    raise NotImplementedError("write your pallas kernel here")


# speedup vs baseline: 1.6592x; 1.6592x over previous
import jax
import jax.numpy as jnp
from jax.experimental import pallas as pl

_B, _C, _K, _D = 4096, 16, 1024, 32
_BT = 512


def _dpq_kernel(x_ref, w_ref, idx_ref, mse_ref):
    for c in range(_C):
        x = x_ref[:, c, :]
        w = w_ref[c]
        resp = jax.lax.dot_general(
            x, w, (((1,), (1,)), ((), ())),
            preferred_element_type=jnp.float32)
        mx = jnp.max(resp, axis=1)
        lane = jax.lax.broadcasted_iota(jnp.int32, resp.shape, 1)
        idx = jnp.min(jnp.where(resp == mx[:, None], lane, _K), axis=1)
        mse_ref[:, c:c + 1] = mx[:, None]
        idx_ref[:, c:c + 1] = (idx + c * _K)[:, None]


def kernel(inputs, centroids):
    idx, mse = pl.pallas_call(
        _dpq_kernel,
        grid=(_B // _BT,),
        in_specs=[
            pl.BlockSpec((_BT, _C, _D), lambda i: (i, 0, 0)),
            pl.BlockSpec((_C, _K, _D), lambda i: (0, 0, 0)),
        ],
        out_specs=[
            pl.BlockSpec((_BT, _C), lambda i: (i, 0)),
            pl.BlockSpec((_BT, _C), lambda i: (i, 0)),
        ],
        out_shape=[
            jax.ShapeDtypeStruct((_B, _C), jnp.int32),
            jax.ShapeDtypeStruct((_B, _C), jnp.float32),
        ],
    )(inputs, centroids)
    return (idx, mse, centroids)


# speedup vs baseline: 2.1898x; 1.3198x over previous
import jax
import jax.numpy as jnp
from jax.experimental import pallas as pl

_B, _C, _K, _D = 4096, 16, 1024, 32
_BT = 512


def _dpq_kernel(x_ref, w_ref, idx_ref, mse_ref):
    lane_rev = (_K - 1) - jax.lax.broadcasted_iota(jnp.int32, (_BT, _K), 1)
    for c in range(_C):
        x = x_ref[:, c, :]
        w = w_ref[c]
        resp = jax.lax.dot_general(
            x, w, (((1,), (1,)), ((), ())),
            preferred_element_type=jnp.float32)
        bits = jax.lax.bitcast_convert_type(resp, jnp.int32)
        packed = (bits & ~(_K - 1)) | lane_rev
        pmax = jnp.max(jax.lax.bitcast_convert_type(packed, jnp.float32),
                       axis=1)
        pbits = jax.lax.bitcast_convert_type(pmax, jnp.int32)
        idx = (_K - 1) - (pbits & (_K - 1))
        mse = jax.lax.bitcast_convert_type(pbits & ~(_K - 1), jnp.float32)
        mse_ref[:, c:c + 1] = mse[:, None]
        idx_ref[:, c:c + 1] = (idx + c * _K)[:, None]


def kernel(inputs, centroids):
    idx, mse = pl.pallas_call(
        _dpq_kernel,
        grid=(_B // _BT,),
        in_specs=[
            pl.BlockSpec((_BT, _C, _D), lambda i: (i, 0, 0)),
            pl.BlockSpec((_C, _K, _D), lambda i: (0, 0, 0)),
        ],
        out_specs=[
            pl.BlockSpec((_BT, _C), lambda i: (i, 0)),
            pl.BlockSpec((_BT, _C), lambda i: (i, 0)),
        ],
        out_shape=[
            jax.ShapeDtypeStruct((_B, _C), jnp.int32),
            jax.ShapeDtypeStruct((_B, _C), jnp.float32),
        ],
    )(inputs, centroids)
    return (idx, mse, centroids)
